# Initial kernel scaffold; baseline (speedup 1.0000x reference)
import jax
import jax.numpy as jnp
from jax.experimental import pallas as pl


def kernel(x, edge_index, edge_attr, W1, b1, W2):
    raise NotImplementedError("write your pallas kernel here")


# speedup vs baseline: 2.9947x; 2.9947x over previous
import functools

import jax
import jax.numpy as jnp
from jax import lax
from jax.experimental import pallas as pl
from jax.experimental.pallas import tpu as pltpu
from jax.experimental.pallas import tpu_sc as plsc

N = 10000
E = 320000
D = 128
DE = 4
NF = 64

NC = 2
NS = 16
LN = 16
NW = NC * NS
EPW = E // NW
C = 80
NCHUNK = EPW // C
G = C // LN
NP = 10240
RPT = NP // NS
ZR = C
ZC = 64


def _sc_body(x_hbm, row_hbm, col_hbm, attrT_hbm, w_hbm,
             m_hbm, num_hbm, cnt_hbm,
             row_v, col_v, attr_v, w_v, src_v, m_v,
             ones_v, eo_v, zc_v, num_s, cnt_s, sem):
    cid = lax.axis_index("c")
    sid = lax.axis_index("s")
    wid = sid * NC + cid

    def fill_src0(r, _):
        for j in range(D // LN):
            src_v[r, pl.ds(j * LN, LN)] = jnp.zeros((LN,), jnp.float32)
        return 0
    lax.fori_loop(0, ZR, fill_src0, 0)

    def fill_zc(r, _):
        zc_v[r, :] = jnp.zeros((LN,), jnp.float32)
        return 0
    lax.fori_loop(0, ZC, fill_zc, 0)

    def fill_ones(r, _):
        ones_v[r, :] = jnp.ones((LN,), jnp.float32)
        return 0
    lax.fori_loop(0, C, fill_ones, 0)

    for q in range(RPT // ZR):
        pltpu.sync_copy(src_v, num_s.at[pl.ds(sid * RPT + q * ZR, ZR)])
    for q in range(RPT // ZC):
        pltpu.sync_copy(zc_v, cnt_s.at[pl.ds(sid * RPT + q * ZC, ZC)])

    pltpu.sync_copy(w_hbm, w_v)

    plsc.subcore_barrier()

    def chunk_body(i, _):
        pltpu.sync_copy(row_hbm.at[wid, i], row_v)
        pltpu.sync_copy(col_hbm.at[wid, i], col_v)
        pltpu.sync_copy(attrT_hbm.at[wid, i], attr_v)
        cp1 = pltpu.async_copy(x_hbm.at[row_v], src_v, sem)
        cp2 = pltpu.async_copy(x_hbm.at[col_v], m_v, sem)
        cp1.wait()
        cp2.wait()

        lane = lax.iota(jnp.int32, LN)
        for g in range(G):
            e0 = g * LN

            def edge_body(e, r2v):
                acc = jnp.zeros((LN,), jnp.float32)
                for j in range(D // LN):
                    s = src_v[e0 + e, pl.ds(j * LN, LN)]
                    t = m_v[e0 + e, pl.ds(j * LN, LN)]
                    d = s - t
                    m_v[e0 + e, pl.ds(j * LN, LN)] = d
                    acc = acc + d * d
                return jnp.where(lane == e, jnp.sum(acc), r2v)
            r2 = lax.fori_loop(0, LN, edge_body,
                               jnp.zeros((LN,), jnp.float32))

            bi = lax.bitcast_convert_type(r2, jnp.int32)
            yi = jnp.int32(0x5F3759DF) - lax.shift_right_logical(bi, 1)
            y = lax.bitcast_convert_type(yi, jnp.float32)
            h2 = 0.5 * r2
            for _u in range(3):
                y = y * (1.5 - h2 * y * y)
            radial = jnp.where(r2 > 0.0, r2 * y, 0.0)

            a1 = attr_v[0, pl.ds(e0, LN)]
            a2 = attr_v[1, pl.ds(e0, LN)]
            a3 = attr_v[2, pl.ds(e0, LN)]
            a4 = attr_v[3, pl.ds(e0, LN)]

            def mlp_body(j, acc):
                w = w_v[pl.ds(j * 8, LN)]
                h = (radial * w[0] + a1 * w[1] + a2 * w[2]
                     + a3 * w[3] + a4 * w[4] + w[5])
                return acc + h * w[6] + jnp.abs(h) * w[7]
            eo = lax.fori_loop(0, NF, mlp_body, jnp.zeros((LN,), jnp.float32))
            eo_v[:] = eo

            def mpass(e, _):
                s = plsc.load_gather(eo_v, [jnp.full((LN,), e, jnp.int32)])
                for j in range(D // LN):
                    m_v[e0 + e, pl.ds(j * LN, LN)] = (
                        m_v[e0 + e, pl.ds(j * LN, LN)] * s)
                return 0
            lax.fori_loop(0, LN, mpass, 0)

        base_e = wid * EPW + i * C
        pltpu.sync_copy(m_v, m_hbm.at[pl.ds(base_e, C)])
        pltpu.sync_copy(m_v, num_s.at[row_v], add=True)
        pltpu.sync_copy(ones_v, cnt_s.at[row_v], add=True)
        return 0

    lax.fori_loop(0, NCHUNK, chunk_body, 0)

    plsc.subcore_barrier()

    for q in range(RPT // ZR):
        pltpu.sync_copy(num_s.at[pl.ds(sid * RPT + q * ZR, ZR)], src_v)
        pltpu.sync_copy(src_v, num_hbm.at[cid, pl.ds(sid * RPT + q * ZR, ZR)])
    for q in range(RPT // ZC):
        pltpu.sync_copy(cnt_s.at[pl.ds(sid * RPT + q * ZC, ZC)], zc_v)
        pltpu.sync_copy(zc_v, cnt_hbm.at[cid, pl.ds(sid * RPT + q * ZC, ZC)])


_sc_kernel = functools.partial(
    pl.kernel,
    out_type=[
        jax.ShapeDtypeStruct((E, D), jnp.float32),
        jax.ShapeDtypeStruct((NC, NP, D), jnp.float32),
        jax.ShapeDtypeStruct((NC, NP, LN), jnp.float32),
    ],
    mesh=plsc.VectorSubcoreMesh(
        core_axis_name="c", subcore_axis_name="s",
        num_cores=NC, num_subcores=NS),
    compiler_params=pltpu.CompilerParams(
        needs_layout_passes=False, use_tc_tiling_on_sc=False),
    scratch_types=[
        pltpu.VMEM((C,), jnp.int32),
        pltpu.VMEM((C,), jnp.int32),
        pltpu.VMEM((DE, C), jnp.float32),
        pltpu.VMEM((NF * 8 + 8,), jnp.float32),
        pltpu.VMEM((C, D), jnp.float32),
        pltpu.VMEM((C, D), jnp.float32),
        pltpu.VMEM((C, LN), jnp.float32),
        pltpu.VMEM((LN,), jnp.float32),
        pltpu.VMEM((ZC, LN), jnp.float32),
        pltpu.VMEM_SHARED((NP, D), jnp.float32),
        pltpu.VMEM_SHARED((NP, LN), jnp.float32),
        pltpu.SemaphoreType.DMA,
    ],
)(_sc_body)


def _combine_body(x_ref, num_ref, cnt_ref, out_ref):
    n = num_ref[0] + num_ref[1]
    c = cnt_ref[0] + cnt_ref[1]
    c0 = c[:, 0:1]
    out_ref[...] = x_ref[...] + n / jnp.maximum(c0, 1.0)


_ROWS_BLK = 1000


def _combine(x, num, cnt):
    grid = N // _ROWS_BLK
    return pl.pallas_call(
        _combine_body,
        grid=(grid,),
        in_specs=[
            pl.BlockSpec((_ROWS_BLK, D), lambda i: (i, 0)),
            pl.BlockSpec((NC, _ROWS_BLK, D), lambda i: (0, i, 0)),
            pl.BlockSpec((NC, _ROWS_BLK, LN), lambda i: (0, i, 0)),
        ],
        out_specs=pl.BlockSpec((_ROWS_BLK, D), lambda i: (i, 0)),
        out_shape=jax.ShapeDtypeStruct((N, D), jnp.float32),
    )(x, num, cnt)


def kernel(x, edge_index, edge_attr, W1, b1, W2):
    row = edge_index[0].reshape(NW, NCHUNK, C)
    col = edge_index[1].reshape(NW, NCHUNK, C)
    attrT = edge_attr.reshape(NW, NCHUNK, C, DE).transpose(0, 1, 3, 2)
    w2f = W2.reshape(NF)
    wvec = jnp.concatenate(
        [W1, b1[:, None], (0.6 * w2f)[:, None], (0.4 * w2f)[:, None]],
        axis=1).reshape(-1)
    wvec = jnp.concatenate([wvec, jnp.zeros((8,), jnp.float32)])
    m_ij, num, cnt = _sc_kernel(x, row, col, attrT, wvec)
    x_out = _combine(x, num, cnt)
    return (x_out, m_ij)
